# baseline (speedup 1.0000x reference)
import numpy as np

import jax
import jax.numpy as jnp
from jax import lax
from jax.experimental import pallas as pl
from jax.experimental.pallas import tpu as pltpu
from jax.experimental.pallas import tpu_sc as plsc

E = 4096
D_IN = 600
D_EDGE = 300
D_HID = 300

NC = 2
NS = 16
NW = NC * NS
EPW = E // NW
ACC_W = 128
ACC_H = 72

_EYE = np.eye(ACC_W, dtype=np.float32)
_ZEROS = np.zeros((ACC_H, ACC_W), np.float32)


def _sc_body(dst1, src1, dst2, src2, dst1c, src1c, dst2c, src2c,
             eye_hbm, zeros_hbm,
             out1, out2, out1c, out2c,
             idx_d, idx_s, oh_v, acc1, acc2, acc1c, acc2c):
    c = lax.axis_index("c")
    s = lax.axis_index("s")
    base = (s * NC + c) * EPW

    @pl.when(s == 0)
    def _zero():
        pltpu.sync_copy(zeros_hbm, acc1)
        pltpu.sync_copy(zeros_hbm, acc2)
        pltpu.sync_copy(zeros_hbm, acc1c)
        pltpu.sync_copy(zeros_hbm, acc2c)

    plsc.subcore_barrier()

    for dstr, srcr, acc, n_src in ((dst1, src1, acc1, 64),
                                   (dst2, src2, acc2, 64),
                                   (dst1c, src1c, acc1c, 128),
                                   (dst2c, src2c, acc2c, 128)):
        pltpu.sync_copy(dstr.at[pl.ds(base, EPW)], idx_d)
        pltpu.sync_copy(srcr.at[pl.ds(base, EPW)], idx_s)
        for k in range(EPW // 16):
            d = idx_d[pl.ds(k * 16, 16)]
            sv = idx_s[pl.ds(k * 16, 16)]
            ok = (d >= 0) & (d < 64) & (sv >= 0) & (sv < n_src)
            idx_d[pl.ds(k * 16, 16)] = jnp.where(ok, d, 64)
            idx_s[pl.ds(k * 16, 16)] = jnp.where(ok, sv, 0)
        pltpu.sync_copy(eye_hbm.at[idx_s], oh_v)
        pltpu.sync_copy(oh_v, acc.at[idx_d], add=True)

    plsc.subcore_barrier()

    @pl.when(s == 0)
    def _writeout():
        pltpu.sync_copy(acc1, out1.at[c])
        pltpu.sync_copy(acc2, out2.at[c])
        pltpu.sync_copy(acc1c, out1c.at[c])
        pltpu.sync_copy(acc2c, out2c.at[c])


def _sc_counts(dst1, src1, dst2, src2, dst1c, src1c, dst2c, src2c):
    mesh = plsc.VectorSubcoreMesh(core_axis_name="c", subcore_axis_name="s")
    f = pl.kernel(
        _sc_body,
        mesh=mesh,
        out_type=[jax.ShapeDtypeStruct((NC, ACC_H, ACC_W), jnp.float32)] * 4,
        scratch_types=[
            pltpu.VMEM((EPW,), jnp.int32),
            pltpu.VMEM((EPW,), jnp.int32),
            pltpu.VMEM((EPW, ACC_W), jnp.float32),
            pltpu.VMEM_SHARED((ACC_H, ACC_W), jnp.float32),
            pltpu.VMEM_SHARED((ACC_H, ACC_W), jnp.float32),
            pltpu.VMEM_SHARED((ACC_H, ACC_W), jnp.float32),
            pltpu.VMEM_SHARED((ACC_H, ACC_W), jnp.float32),
        ],
    )
    return f(dst1, src1, dst2, src2, dst1c, src1c, dst2c, src2c,
             jnp.asarray(_EYE), jnp.asarray(_ZEROS))


def _ea_proj(dst, ea, We):
    od = jnp.where(dst[:, None] == lax.broadcasted_iota(jnp.int32, (E, 64), 1),
                   1.0, 0.0)
    ea_sum = lax.dot_general(od, ea, (((0,), (0,)), ((), ())),
                             preferred_element_type=jnp.float32)
    return jnp.dot(ea_sum, We, preferred_element_type=jnp.float32)


def _tca_body(x1_r, x2_r, dst1_r, dst2_r, dst1c_r, dst2c_r,
              ea1_r, ea2_r, ea1c_r, ea2c_r,
              tsa_Wn, tsa_We, gsa_Wn, gsa_We, tca_We, gca_We,
              px1_r, px2_r, e1_r, e2_r, e1c_r, e2c_r):
    px1_r[...] = jnp.dot(x1_r[...], tsa_Wn[...],
                         preferred_element_type=jnp.float32)
    px2_r[...] = jnp.dot(x2_r[...], gsa_Wn[...],
                         preferred_element_type=jnp.float32)
    e1_r[...] = _ea_proj(dst1_r[...], ea1_r[...], tsa_We[...])
    e2_r[...] = _ea_proj(dst2_r[...], ea2_r[...], gsa_We[...])
    e1c_r[...] = _ea_proj(dst1c_r[...], ea1c_r[...], tca_We[...])
    e2c_r[...] = _ea_proj(dst2c_r[...], ea2c_r[...], gca_We[...])


def _tca_call(*args):
    return pl.pallas_call(
        _tca_body,
        out_shape=[jax.ShapeDtypeStruct((64, D_HID), jnp.float32)] * 6,
    )(*args)


def _finish(a_part, px, e_proj, bn, be, Wo, bo, n_src):
    a = jnp.sum(a_part, axis=0)[:64, :n_src]
    deg = jnp.sum(a, axis=1)
    eye = jnp.where(lax.broadcasted_iota(jnp.int32, (64, n_src), 0)
                    == lax.broadcasted_iota(jnp.int32, (64, n_src), 1),
                    1.0, 0.0)
    m = jnp.dot(a + deg[:, None] * eye, px, preferred_element_type=jnp.float32)
    agg = m + e_proj + deg[:, None] * (2.0 * bn + be)[None, :]
    return jnp.dot(agg, Wo, preferred_element_type=jnp.float32) + bo[None, :]


def _tcb_body(a1_r, a2_r, a1c_r, a2c_r,
              px1_r, px2_r, e1_r, e2_r, e1c_r, e2c_r,
              tsa_bn, tsa_be, tsa_Wo, tsa_bo,
              gsa_bn, gsa_be, gsa_Wo, gsa_bo,
              tca_Wn, tca_bn, tca_be, tca_Wo, tca_bo,
              gca_Wn, gca_bn, gca_be, gca_Wo, gca_bo,
              o1_r, o2_r):
    y1 = _finish(a1_r[...], px1_r[...], e1_r[...],
                 tsa_bn[...], tsa_be[...], tsa_Wo[...], tsa_bo[...], 64)
    y2 = _finish(a2_r[...], px2_r[...], e2_r[...],
                 gsa_bn[...], gsa_be[...], gsa_Wo[...], gsa_bo[...], 64)
    px1c = jnp.dot(jnp.concatenate([y1, y2], axis=0), tca_Wn[...],
                   preferred_element_type=jnp.float32)
    px2c = jnp.dot(jnp.concatenate([y2, y1], axis=0), gca_Wn[...],
                   preferred_element_type=jnp.float32)
    o1_r[...] = _finish(a1c_r[...], px1c, e1c_r[...],
                        tca_bn[...], tca_be[...], tca_Wo[...], tca_bo[...], 128)
    o2_r[...] = _finish(a2c_r[...], px2c, e2c_r[...],
                        gca_bn[...], gca_be[...], gca_Wo[...], gca_bo[...], 128)


def _tcb_call(*args):
    return pl.pallas_call(
        _tcb_body,
        out_shape=[jax.ShapeDtypeStruct((64, D_IN), jnp.float32)] * 2,
    )(*args)


def kernel(x_1, x_2, edge_index_1, edge_index_2, edge_attr_1, edge_attr_2,
           edge_index_1_cross, edge_attr_1_cross, edge_index_2_cross,
           edge_attr_2_cross,
           tsa_Wn, tsa_bn, tsa_We, tsa_be, tsa_Wo, tsa_bo,
           gsa_Wn, gsa_bn, gsa_We, gsa_be, gsa_Wo, gsa_bo,
           tca_Wn, tca_bn, tca_We, tca_be, tca_Wo, tca_bo,
           gca_Wn, gca_bn, gca_We, gca_be, gca_Wo, gca_bo):
    dst1 = edge_index_1[1].astype(jnp.int32)
    src1 = edge_index_1[0].astype(jnp.int32)
    dst2 = edge_index_2[1].astype(jnp.int32)
    src2 = edge_index_2[0].astype(jnp.int32)
    dst1c = edge_index_1_cross[1].astype(jnp.int32)
    src1c = edge_index_1_cross[0].astype(jnp.int32)
    dst2c = edge_index_2_cross[1].astype(jnp.int32)
    src2c = edge_index_2_cross[0].astype(jnp.int32)

    a1, a2, a1c, a2c = _sc_counts(dst1, src1, dst2, src2,
                                  dst1c, src1c, dst2c, src2c)

    px1, px2, e1, e2, e1c, e2c = _tca_call(
        x_1, x_2, dst1, dst2, dst1c, dst2c,
        edge_attr_1, edge_attr_2, edge_attr_1_cross, edge_attr_2_cross,
        tsa_Wn, tsa_We, gsa_Wn, gsa_We, tca_We, gca_We)

    o1, o2 = _tcb_call(
        a1, a2, a1c, a2c, px1, px2, e1, e2, e1c, e2c,
        tsa_bn, tsa_be, tsa_Wo, tsa_bo,
        gsa_bn, gsa_be, gsa_Wo, gsa_bo,
        tca_Wn, tca_bn, tca_be, tca_Wo, tca_bo,
        gca_Wn, gca_bn, gca_be, gca_Wo, gca_bo)
    return (o1, o2)


# speedup vs baseline: 4.4820x; 4.4820x over previous
import numpy as np

import jax
import jax.numpy as jnp
from jax import lax
from jax.experimental import pallas as pl
from jax.experimental.pallas import tpu as pltpu
from jax.experimental.pallas import tpu_sc as plsc

E = 4096
D_IN = 600
D_EDGE = 300
D_HID = 300

NC = 2
NS = 16
NW = NC * NS
EPW = E // NW
ACC_W = 128
ACC_H = 128

_EYE = np.eye(ACC_W, dtype=np.float32)
_ZEROS = np.zeros((ACC_H, ACC_W), np.float32)


def _sc_body(dst1, src1, dst2, src2, dst1c, src1c, dst2c, src2c,
             eye_hbm, zeros_hbm,
             out1, out2, out1c, out2c,
             idx_d, idx_s, oh_v, acc1, acc2, acc1c, acc2c):
    c = lax.axis_index("c")
    s = lax.axis_index("s")
    base = (s * NC + c) * EPW

    @pl.when(s == 0)
    def _zero():
        pltpu.sync_copy(zeros_hbm, acc1)
        pltpu.sync_copy(zeros_hbm, acc2)
        pltpu.sync_copy(zeros_hbm, acc1c)
        pltpu.sync_copy(zeros_hbm, acc2c)

    plsc.subcore_barrier()

    for dstr, srcr, acc, n_src in ((dst1, src1, acc1, 64),
                                   (dst2, src2, acc2, 64),
                                   (dst1c, src1c, acc1c, 128),
                                   (dst2c, src2c, acc2c, 128)):
        pltpu.sync_copy(dstr.at[pl.ds(base, EPW)], idx_d)
        pltpu.sync_copy(srcr.at[pl.ds(base, EPW)], idx_s)
        for k in range(EPW // 16):
            d = idx_d[pl.ds(k * 16, 16)]
            sv = idx_s[pl.ds(k * 16, 16)]
            ok = (d >= 0) & (d < 128) & (sv >= 0) & (sv < n_src)
            idx_d[pl.ds(k * 16, 16)] = jnp.where(ok, d, 127)
            idx_s[pl.ds(k * 16, 16)] = jnp.where(ok, sv, 0)
        pltpu.sync_copy(eye_hbm.at[idx_s], oh_v)
        pltpu.sync_copy(oh_v, acc.at[idx_d], add=True)

    plsc.subcore_barrier()

    @pl.when(s == 0)
    def _writeout():
        pltpu.sync_copy(acc1, out1.at[c])
        pltpu.sync_copy(acc2, out2.at[c])
        pltpu.sync_copy(acc1c, out1c.at[c])
        pltpu.sync_copy(acc2c, out2c.at[c])


def _sc_counts(dst1, src1, dst2, src2, dst1c, src1c, dst2c, src2c):
    mesh = plsc.VectorSubcoreMesh(core_axis_name="c", subcore_axis_name="s")
    f = pl.kernel(
        _sc_body,
        mesh=mesh,
        out_type=[jax.ShapeDtypeStruct((NC, ACC_H, ACC_W), jnp.float32)] * 4,
        scratch_types=[
            pltpu.VMEM((EPW,), jnp.int32),
            pltpu.VMEM((EPW,), jnp.int32),
            pltpu.VMEM((EPW, ACC_W), jnp.float32),
            pltpu.VMEM_SHARED((ACC_H, ACC_W), jnp.float32),
            pltpu.VMEM_SHARED((ACC_H, ACC_W), jnp.float32),
            pltpu.VMEM_SHARED((ACC_H, ACC_W), jnp.float32),
            pltpu.VMEM_SHARED((ACC_H, ACC_W), jnp.float32),
        ],
    )
    return f(dst1, src1, dst2, src2, dst1c, src1c, dst2c, src2c,
             jnp.asarray(_EYE), jnp.asarray(_ZEROS))


def _ea_proj(dst, ea, We):
    od = jnp.where(dst[:, None] == lax.broadcasted_iota(jnp.int32, (E, 64), 1),
                   1.0, 0.0)
    ea_sum = lax.dot_general(od, ea, (((0,), (0,)), ((), ())),
                             preferred_element_type=jnp.float32)
    return jnp.dot(ea_sum, We, preferred_element_type=jnp.float32)


def _tca_body(x1_r, x2_r, dst1_r, dst2_r, dst1c_r, dst2c_r,
              ea1_r, ea2_r, ea1c_r, ea2c_r,
              tsa_Wn, tsa_We, gsa_Wn, gsa_We, tca_We, gca_We,
              px1_r, px2_r, e1_r, e2_r, e1c_r, e2c_r):
    px1_r[...] = jnp.dot(x1_r[...], tsa_Wn[...],
                         preferred_element_type=jnp.float32)
    px2_r[...] = jnp.dot(x2_r[...], gsa_Wn[...],
                         preferred_element_type=jnp.float32)
    e1_r[...] = _ea_proj(dst1_r[...], ea1_r[...], tsa_We[...])
    e2_r[...] = _ea_proj(dst2_r[...], ea2_r[...], gsa_We[...])
    e1c_r[...] = _ea_proj(dst1c_r[...], ea1c_r[...], tca_We[...])
    e2c_r[...] = _ea_proj(dst2c_r[...], ea2c_r[...], gca_We[...])


def _tca_call(*args):
    return pl.pallas_call(
        _tca_body,
        out_shape=[jax.ShapeDtypeStruct((64, D_HID), jnp.float32)] * 6,
    )(*args)


def _finish(a_part, px, e_proj, bn, be, Wo, bo, n_src):
    a = jnp.sum(a_part, axis=0)[:64, :n_src]
    deg = jnp.sum(a, axis=1)
    eye = jnp.where(lax.broadcasted_iota(jnp.int32, (64, n_src), 0)
                    == lax.broadcasted_iota(jnp.int32, (64, n_src), 1),
                    1.0, 0.0)
    m = jnp.dot(a + deg[:, None] * eye, px, preferred_element_type=jnp.float32)
    agg = m + e_proj + deg[:, None] * (2.0 * bn + be)[None, :]
    return jnp.dot(agg, Wo, preferred_element_type=jnp.float32) + bo[None, :]


def _tcb_body(a1_r, a2_r, a1c_r, a2c_r,
              px1_r, px2_r, e1_r, e2_r, e1c_r, e2c_r,
              tsa_bn, tsa_be, tsa_Wo, tsa_bo,
              gsa_bn, gsa_be, gsa_Wo, gsa_bo,
              tca_Wn, tca_bn, tca_be, tca_Wo, tca_bo,
              gca_Wn, gca_bn, gca_be, gca_Wo, gca_bo,
              o1_r, o2_r):
    y1 = _finish(a1_r[...], px1_r[...], e1_r[...],
                 tsa_bn[...], tsa_be[...], tsa_Wo[...], tsa_bo[...], 64)
    y2 = _finish(a2_r[...], px2_r[...], e2_r[...],
                 gsa_bn[...], gsa_be[...], gsa_Wo[...], gsa_bo[...], 64)
    px1c = jnp.dot(jnp.concatenate([y1, y2], axis=0), tca_Wn[...],
                   preferred_element_type=jnp.float32)
    px2c = jnp.dot(jnp.concatenate([y2, y1], axis=0), gca_Wn[...],
                   preferred_element_type=jnp.float32)
    o1_r[...] = _finish(a1c_r[...], px1c, e1c_r[...],
                        tca_bn[...], tca_be[...], tca_Wo[...], tca_bo[...], 128)
    o2_r[...] = _finish(a2c_r[...], px2c, e2c_r[...],
                        gca_bn[...], gca_be[...], gca_Wo[...], gca_bo[...], 128)


def _tcb_call(*args):
    return pl.pallas_call(
        _tcb_body,
        out_shape=[jax.ShapeDtypeStruct((64, D_IN), jnp.float32)] * 2,
    )(*args)


def kernel(x_1, x_2, edge_index_1, edge_index_2, edge_attr_1, edge_attr_2,
           edge_index_1_cross, edge_attr_1_cross, edge_index_2_cross,
           edge_attr_2_cross,
           tsa_Wn, tsa_bn, tsa_We, tsa_be, tsa_Wo, tsa_bo,
           gsa_Wn, gsa_bn, gsa_We, gsa_be, gsa_Wo, gsa_bo,
           tca_Wn, tca_bn, tca_We, tca_be, tca_Wo, tca_bo,
           gca_Wn, gca_bn, gca_We, gca_be, gca_Wo, gca_bo):
    dst1 = edge_index_1[1].astype(jnp.int32)
    src1 = edge_index_1[0].astype(jnp.int32)
    dst2 = edge_index_2[1].astype(jnp.int32)
    src2 = edge_index_2[0].astype(jnp.int32)
    dst1c = edge_index_1_cross[1].astype(jnp.int32)
    src1c = edge_index_1_cross[0].astype(jnp.int32)
    dst2c = edge_index_2_cross[1].astype(jnp.int32)
    src2c = edge_index_2_cross[0].astype(jnp.int32)

    a1, a2, a1c, a2c = _sc_counts(dst1, src1, dst2, src2,
                                  dst1c, src1c, dst2c, src2c)

    px1, px2, e1, e2, e1c, e2c = _tca_call(
        x_1, x_2, dst1, dst2, dst1c, dst2c,
        edge_attr_1, edge_attr_2, edge_attr_1_cross, edge_attr_2_cross,
        tsa_Wn, tsa_We, gsa_Wn, gsa_We, tca_We, gca_We)

    o1, o2 = _tcb_call(
        a1, a2, a1c, a2c, px1, px2, e1, e2, e1c, e2c,
        tsa_bn, tsa_be, tsa_Wo, tsa_bo,
        gsa_bn, gsa_be, gsa_Wo, gsa_bo,
        tca_Wn, tca_bn, tca_be, tca_Wo, tca_bo,
        gca_Wn, gca_bn, gca_be, gca_Wo, gca_bo)
    return (o1, o2)


# speedup vs baseline: 4.8958x; 1.0923x over previous
import numpy as np

import jax
import jax.numpy as jnp
from jax import lax
from jax.experimental import pallas as pl
from jax.experimental.pallas import tpu as pltpu
from jax.experimental.pallas import tpu_sc as plsc

E = 4096
D_IN = 600
D_EDGE = 300
D_HID = 300

NC = 2
NS = 16
NW = NC * NS
EPW = E // NW
ACC_W = 128
ACC_H = 128

_EYE = np.eye(ACC_W, dtype=np.float32)
_ZEROS = np.zeros((ACC_H, ACC_W), np.float32)


def _sc_body(dst1, src1, dst2, src2,
             eye_hbm, zeros_hbm,
             out1, out2,
             idx_d, idx_s, oh_v, acc1, acc2):
    c = lax.axis_index("c")
    s = lax.axis_index("s")
    base = (s * NC + c) * EPW

    @pl.when(s == 0)
    def _zero():
        pltpu.sync_copy(zeros_hbm, acc1)
        pltpu.sync_copy(zeros_hbm, acc2)

    plsc.subcore_barrier()

    for dstr, srcr, acc, n_src in ((dst1, src1, acc1, 64),
                                   (dst2, src2, acc2, 64)):
        pltpu.sync_copy(dstr.at[pl.ds(base, EPW)], idx_d)
        pltpu.sync_copy(srcr.at[pl.ds(base, EPW)], idx_s)
        for k in range(EPW // 16):
            d = idx_d[pl.ds(k * 16, 16)]
            sv = idx_s[pl.ds(k * 16, 16)]
            ok = (d >= 0) & (d < 128) & (sv >= 0) & (sv < n_src)
            idx_d[pl.ds(k * 16, 16)] = jnp.where(ok, d, 127)
            idx_s[pl.ds(k * 16, 16)] = jnp.where(ok, sv, 0)
        pltpu.sync_copy(eye_hbm.at[idx_s], oh_v)
        pltpu.sync_copy(oh_v, acc.at[idx_d], add=True)

    plsc.subcore_barrier()

    @pl.when(s == 0)
    def _writeout():
        pltpu.sync_copy(acc1, out1.at[c])
        pltpu.sync_copy(acc2, out2.at[c])


def _sc_counts(dst1, src1, dst2, src2):
    mesh = plsc.VectorSubcoreMesh(core_axis_name="c", subcore_axis_name="s")
    f = pl.kernel(
        _sc_body,
        mesh=mesh,
        out_type=[jax.ShapeDtypeStruct((NC, ACC_H, ACC_W), jnp.float32)] * 2,
        scratch_types=[
            pltpu.VMEM((EPW,), jnp.int32),
            pltpu.VMEM((EPW,), jnp.int32),
            pltpu.VMEM((EPW, ACC_W), jnp.float32),
            pltpu.VMEM_SHARED((ACC_H, ACC_W), jnp.float32),
            pltpu.VMEM_SHARED((ACC_H, ACC_W), jnp.float32),
        ],
    )
    return f(dst1, src1, dst2, src2, jnp.asarray(_EYE), jnp.asarray(_ZEROS))


def _ea_proj(dst, ea, We):
    od = jnp.where(dst[:, None] == lax.broadcasted_iota(jnp.int32, (E, 64), 1),
                   1.0, 0.0)
    ea_sum = lax.dot_general(od, ea, (((0,), (0,)), ((), ())),
                             preferred_element_type=jnp.float32)
    return jnp.dot(ea_sum, We, preferred_element_type=jnp.float32)


def _count_mm(dst, src):
    od = jnp.where(dst[:, None] == lax.broadcasted_iota(jnp.int32, (E, 64), 1),
                   1.0, 0.0)
    os_ = jnp.where(src[:, None] == lax.broadcasted_iota(jnp.int32, (E, 128), 1),
                    1.0, 0.0)
    return lax.dot_general(od, os_, (((0,), (0,)), ((), ())),
                           preferred_element_type=jnp.float32)


def _tca_body(x1_r, x2_r, dst1_r, dst2_r, dst1c_r, dst2c_r,
              src1c_r, src2c_r,
              ea1_r, ea2_r, ea1c_r, ea2c_r,
              tsa_Wn, tsa_We, gsa_Wn, gsa_We, tca_We, gca_We,
              px1_r, px2_r, e1_r, e2_r, e1c_r, e2c_r, a1c_r, a2c_r):
    px1_r[...] = jnp.dot(x1_r[...], tsa_Wn[...],
                         preferred_element_type=jnp.float32)
    px2_r[...] = jnp.dot(x2_r[...], gsa_Wn[...],
                         preferred_element_type=jnp.float32)
    e1_r[...] = _ea_proj(dst1_r[...], ea1_r[...], tsa_We[...])
    e2_r[...] = _ea_proj(dst2_r[...], ea2_r[...], gsa_We[...])
    e1c_r[...] = _ea_proj(dst1c_r[...], ea1c_r[...], tca_We[...])
    e2c_r[...] = _ea_proj(dst2c_r[...], ea2c_r[...], gca_We[...])
    a1c_r[...] = _count_mm(dst1c_r[...], src1c_r[...])
    a2c_r[...] = _count_mm(dst2c_r[...], src2c_r[...])


def _tca_call(*args):
    return pl.pallas_call(
        _tca_body,
        out_shape=[jax.ShapeDtypeStruct((64, D_HID), jnp.float32)] * 6
        + [jax.ShapeDtypeStruct((64, 128), jnp.float32)] * 2,
    )(*args)


def _finish(a, px, e_proj, bn, be, Wo, bo, n_src):
    deg = jnp.sum(a, axis=1)
    eye = jnp.where(lax.broadcasted_iota(jnp.int32, (64, n_src), 0)
                    == lax.broadcasted_iota(jnp.int32, (64, n_src), 1),
                    1.0, 0.0)
    m = jnp.dot(a + deg[:, None] * eye, px, preferred_element_type=jnp.float32)
    agg = m + e_proj + deg[:, None] * (2.0 * bn + be)[None, :]
    return jnp.dot(agg, Wo, preferred_element_type=jnp.float32) + bo[None, :]


def _tcb_body(a1_r, a2_r, a1c_r, a2c_r,
              px1_r, px2_r, e1_r, e2_r, e1c_r, e2c_r,
              tsa_bn, tsa_be, tsa_Wo, tsa_bo,
              gsa_bn, gsa_be, gsa_Wo, gsa_bo,
              tca_Wn, tca_bn, tca_be, tca_Wo, tca_bo,
              gca_Wn, gca_bn, gca_be, gca_Wo, gca_bo,
              o1_r, o2_r):
    y1 = _finish(jnp.sum(a1_r[...], axis=0)[:64, :64], px1_r[...], e1_r[...],
                 tsa_bn[...], tsa_be[...], tsa_Wo[...], tsa_bo[...], 64)
    y2 = _finish(jnp.sum(a2_r[...], axis=0)[:64, :64], px2_r[...], e2_r[...],
                 gsa_bn[...], gsa_be[...], gsa_Wo[...], gsa_bo[...], 64)
    px1c = jnp.dot(jnp.concatenate([y1, y2], axis=0), tca_Wn[...],
                   preferred_element_type=jnp.float32)
    px2c = jnp.dot(jnp.concatenate([y2, y1], axis=0), gca_Wn[...],
                   preferred_element_type=jnp.float32)
    o1_r[...] = _finish(a1c_r[...], px1c, e1c_r[...],
                        tca_bn[...], tca_be[...], tca_Wo[...], tca_bo[...], 128)
    o2_r[...] = _finish(a2c_r[...], px2c, e2c_r[...],
                        gca_bn[...], gca_be[...], gca_Wo[...], gca_bo[...], 128)


def _tcb_call(*args):
    return pl.pallas_call(
        _tcb_body,
        out_shape=[jax.ShapeDtypeStruct((64, D_IN), jnp.float32)] * 2,
    )(*args)


def kernel(x_1, x_2, edge_index_1, edge_index_2, edge_attr_1, edge_attr_2,
           edge_index_1_cross, edge_attr_1_cross, edge_index_2_cross,
           edge_attr_2_cross,
           tsa_Wn, tsa_bn, tsa_We, tsa_be, tsa_Wo, tsa_bo,
           gsa_Wn, gsa_bn, gsa_We, gsa_be, gsa_Wo, gsa_bo,
           tca_Wn, tca_bn, tca_We, tca_be, tca_Wo, tca_bo,
           gca_Wn, gca_bn, gca_We, gca_be, gca_Wo, gca_bo):
    dst1 = edge_index_1[1].astype(jnp.int32)
    src1 = edge_index_1[0].astype(jnp.int32)
    dst2 = edge_index_2[1].astype(jnp.int32)
    src2 = edge_index_2[0].astype(jnp.int32)
    dst1c = edge_index_1_cross[1].astype(jnp.int32)
    src1c = edge_index_1_cross[0].astype(jnp.int32)
    dst2c = edge_index_2_cross[1].astype(jnp.int32)
    src2c = edge_index_2_cross[0].astype(jnp.int32)

    a1, a2 = _sc_counts(dst1, src1, dst2, src2)

    px1, px2, e1, e2, e1c, e2c, a1c, a2c = _tca_call(
        x_1, x_2, dst1, dst2, dst1c, dst2c, src1c, src2c,
        edge_attr_1, edge_attr_2, edge_attr_1_cross, edge_attr_2_cross,
        tsa_Wn, tsa_We, gsa_Wn, gsa_We, tca_We, gca_We)

    o1, o2 = _tcb_call(
        a1, a2, a1c, a2c, px1, px2, e1, e2, e1c, e2c,
        tsa_bn, tsa_be, tsa_Wo, tsa_bo,
        gsa_bn, gsa_be, gsa_Wo, gsa_bo,
        tca_Wn, tca_bn, tca_be, tca_Wo, tca_bo,
        gca_Wn, gca_bn, gca_be, gca_Wo, gca_bo)
    return (o1, o2)
